# Initial kernel scaffold; baseline (speedup 1.0000x reference)
import jax
import jax.numpy as jnp
from jax.experimental import pallas as pl


def kernel(x):
    raise NotImplementedError("write your pallas kernel here")


# speedup vs baseline: 1.3492x; 1.3492x over previous
import functools

import jax
import jax.numpy as jnp
from jax import lax
from jax.experimental import pallas as pl
from jax.experimental.pallas import tpu as pltpu
from jax.experimental.pallas import tpu_sc as plsc

_SEED_NUM = 64
_BANDWIDTH = 11.5
_ITERATION = 5

_N = 100000
_D = 64
_C = 10000
_NB = _N // _C


def _iter_body(c0t_ref, x_ref, c_out_ref, ct_out_ref, cpt_out_ref,
               ct, sums_t, counts):
    it = pl.program_id(0)
    j = pl.program_id(1)

    @pl.when(jnp.logical_and(it == 0, j == 0))
    def _init():
        ct[...] = c0t_ref[...]
        sums_t[...] = jnp.zeros_like(sums_t)
        counts[...] = jnp.zeros_like(counts)

    @pl.when(jnp.logical_and(it > 0, j == 0))
    def _advance():
        cnt = jnp.maximum(counts[...], 1.0)
        ct[...] = sums_t[...] / cnt
        sums_t[...] = jnp.zeros_like(sums_t)
        counts[...] = jnp.zeros_like(counts)

    c = ct[...]
    xj = x_ref[...]
    c2 = jnp.sum(c * c, axis=0, keepdims=True)
    x2 = jnp.sum(xj * xj, axis=1, keepdims=True)
    cx = lax.dot_general(xj, c, (((1,), (0,)), ((), ())),
                         preferred_element_type=jnp.float32)
    d2 = x2 + c2 - 2.0 * cx
    dis = jnp.sqrt(jnp.maximum(d2, 0.0))
    maskf = (dis < _BANDWIDTH).astype(jnp.float32)
    sums_t[...] += lax.dot_general(xj, maskf, (((0,), (0,)), ((), ())),
                                   preferred_element_type=jnp.float32)
    counts[...] += jnp.sum(maskf, axis=0, keepdims=True)

    @pl.when(jnp.logical_and(it == _ITERATION - 1, j == _NB - 1))
    def _finish():
        cnt = jnp.maximum(counts[...], 1.0)
        ctf = sums_t[...] / cnt
        c_out_ref[...] = ctf.T
        ct_out_ref[...] = ctf
        cpt_out_ref[...] = ct[...]


def _final_body(cpt_ref, cnt_ref, x_ref, mask_ref, idx_out_ref,
                run_min, run_idx):
    j = pl.program_id(0)

    @pl.when(j == 0)
    def _init():
        run_min[...] = jnp.full_like(run_min, jnp.inf)
        run_idx[...] = jnp.zeros_like(run_idx)

    cp = cpt_ref[...]
    cn = cnt_ref[...]
    xj = x_ref[...]
    x2 = jnp.sum(xj * xj, axis=1, keepdims=True)

    c2p = jnp.sum(cp * cp, axis=0, keepdims=True)
    cxp = lax.dot_general(xj, cp, (((1,), (0,)), ((), ())),
                          preferred_element_type=jnp.float32)
    d2p = x2 + c2p - 2.0 * cxp
    disp = jnp.sqrt(jnp.maximum(d2p, 0.0))
    m = disp < _BANDWIDTH
    mask_ref[...] = m

    c2n = jnp.sum(cn * cn, axis=0, keepdims=True)
    cxn = lax.dot_general(xj, cn, (((1,), (0,)), ((), ())),
                          preferred_element_type=jnp.float32)
    d2n = x2 + c2n - 2.0 * cxn
    disn = jnp.sqrt(jnp.maximum(d2n, 0.0))
    vals = jnp.where(m, disn, jnp.inf)

    bmin = jnp.min(vals, axis=0, keepdims=True)
    ri = lax.broadcasted_iota(jnp.int32, vals.shape, 0)
    cand = jnp.where(vals == bmin, ri, _C)
    bidx = jnp.min(cand, axis=0, keepdims=True)

    upd = bmin < run_min[...]
    run_idx[...] = jnp.where(upd, j * _C + bidx, run_idx[...])
    run_min[...] = jnp.where(upd, bmin, run_min[...])

    @pl.when(j == _NB - 1)
    def _finish():
        idx_out_ref[...] = run_idx[...]


_GATHER_ROWS_PER_WORKER = 8
_GATHER_WORKERS = _SEED_NUM // _GATHER_ROWS_PER_WORKER


def _sc_gather(x, idx):
    info = plsc.get_sparse_core_info()
    nc = info.num_cores
    mesh = plsc.VectorSubcoreMesh(core_axis_name="c", subcore_axis_name="s")

    @functools.partial(
        pl.kernel, mesh=mesh,
        out_type=jax.ShapeDtypeStruct((_SEED_NUM, _D), jnp.float32),
        compiler_params=pltpu.CompilerParams(use_tc_tiling_on_sc=False),
        scratch_types=[
            pltpu.VMEM((_GATHER_ROWS_PER_WORKER,), jnp.int32),
            pltpu.VMEM((_GATHER_ROWS_PER_WORKER, _D), jnp.float32),
            pltpu.SemaphoreType.DMA,
        ],
    )
    def k(table_hbm, idx_hbm, out_hbm, idx_v, rows_v, sem):
        wid = lax.axis_index("s") * nc + lax.axis_index("c")

        @pl.when(wid < _GATHER_WORKERS)
        def _():
            base = wid * _GATHER_ROWS_PER_WORKER
            pltpu.sync_copy(idx_hbm.at[pl.ds(base, _GATHER_ROWS_PER_WORKER)],
                            idx_v)
            pltpu.async_copy(table_hbm.at[idx_v], rows_v, sem).wait()
            pltpu.sync_copy(rows_v,
                            out_hbm.at[pl.ds(base, _GATHER_ROWS_PER_WORKER)])

    return k(x, idx)


def kernel(x):
    with jax.ensure_compile_time_eval():
        idx0 = jax.random.permutation(jax.random.key(1), _N)[:_SEED_NUM]
    c0t = x[idx0, :].T

    centers, cnt_t, cpt_t = pl.pallas_call(
        _iter_body,
        grid=(_ITERATION, _NB),
        in_specs=[
            pl.BlockSpec((_D, _SEED_NUM), lambda it, j: (0, 0)),
            pl.BlockSpec((_C, _D), lambda it, j: (j, 0)),
        ],
        out_specs=[
            pl.BlockSpec((_SEED_NUM, _D), lambda it, j: (0, 0)),
            pl.BlockSpec((_D, _SEED_NUM), lambda it, j: (0, 0)),
            pl.BlockSpec((_D, _SEED_NUM), lambda it, j: (0, 0)),
        ],
        out_shape=[
            jax.ShapeDtypeStruct((_SEED_NUM, _D), jnp.float32),
            jax.ShapeDtypeStruct((_D, _SEED_NUM), jnp.float32),
            jax.ShapeDtypeStruct((_D, _SEED_NUM), jnp.float32),
        ],
        scratch_shapes=[
            pltpu.VMEM((_D, _SEED_NUM), jnp.float32),
            pltpu.VMEM((_D, _SEED_NUM), jnp.float32),
            pltpu.VMEM((1, _SEED_NUM), jnp.float32),
        ],
    )(c0t, x)

    mask_t, idx2d = pl.pallas_call(
        _final_body,
        grid=(_NB,),
        in_specs=[
            pl.BlockSpec((_D, _SEED_NUM), lambda j: (0, 0)),
            pl.BlockSpec((_D, _SEED_NUM), lambda j: (0, 0)),
            pl.BlockSpec((_C, _D), lambda j: (j, 0)),
        ],
        out_specs=[
            pl.BlockSpec((_C, _SEED_NUM), lambda j: (j, 0)),
            pl.BlockSpec((1, _SEED_NUM), lambda j: (0, 0)),
        ],
        out_shape=[
            jax.ShapeDtypeStruct((_N, _SEED_NUM), jnp.bool_),
            jax.ShapeDtypeStruct((1, _SEED_NUM), jnp.int32),
        ],
        scratch_shapes=[
            pltpu.VMEM((1, _SEED_NUM), jnp.float32),
            pltpu.VMEM((1, _SEED_NUM), jnp.int32),
        ],
    )(cpt_t, cnt_t, x)

    mask = mask_t.T
    index_similar = idx2d.reshape(_SEED_NUM)
    center_similar = _sc_gather(x, index_similar)
    return centers, mask, center_similar, index_similar


# speedup vs baseline: 2.2754x; 1.6865x over previous
import functools

import jax
import jax.numpy as jnp
from jax import lax
from jax.experimental import pallas as pl
from jax.experimental.pallas import tpu as pltpu
from jax.experimental.pallas import tpu_sc as plsc

_SEED_NUM = 64
_BANDWIDTH = 11.5
_BW2 = _BANDWIDTH * _BANDWIDTH
_ITERATION = 5

_N = 100000
_D = 64
_C = 12800
_NB = 8
_NPAD = _NB * _C
_PAD_VAL = 1.0e4


def _prep_body(x_ref, xt_ref, thr_ref):
    j = pl.program_id(0)
    xb = x_ref[...]
    xt = xb.T
    lane = lax.broadcasted_iota(jnp.int32, (1, _C), 1)
    valid = lane < (_N - j * _C)
    xt = jnp.where(valid, xt, _PAD_VAL)
    xt_ref[...] = xt
    x2 = jnp.sum(xt * xt, axis=0, keepdims=True)
    thr_ref[...] = (_BW2 - x2).reshape(1, 1, _C)


def _iter_body(centers0_ref, xt_ref, thr_ref, c_out_ref, cprev_out_ref,
               cur, sums, counts):
    it = pl.program_id(0)
    j = pl.program_id(1)

    @pl.when(jnp.logical_and(it == 0, j == 0))
    def _init():
        cur[...] = centers0_ref[...]
        sums[...] = jnp.zeros_like(sums)
        counts[...] = jnp.zeros_like(counts)

    @pl.when(jnp.logical_and(it > 0, j == 0))
    def _advance():
        cnt = jnp.maximum(counts[...], 1.0)
        cur[...] = sums[...] / cnt
        sums[...] = jnp.zeros_like(sums)
        counts[...] = jnp.zeros_like(counts)

    c = cur[...]
    xb = xt_ref[...]
    thr = thr_ref[...].reshape(1, _C)
    c2 = jnp.sum(c * c, axis=1, keepdims=True)
    mm = lax.dot_general(-2.0 * c, xb, (((1,), (0,)), ((), ())),
                         preferred_element_type=jnp.float32)
    maskf = ((mm + c2) < thr).astype(jnp.float32)
    sums[...] += lax.dot_general(maskf, xb, (((1,), (1,)), ((), ())),
                                 preferred_element_type=jnp.float32)
    ones_col = jnp.ones((_C, 1), dtype=jnp.float32)
    counts[...] += lax.dot_general(maskf, ones_col, (((1,), (0,)), ((), ())),
                                   preferred_element_type=jnp.float32)

    @pl.when(jnp.logical_and(it == _ITERATION - 1, j == _NB - 1))
    def _finish():
        cnt = jnp.maximum(counts[...], 1.0)
        c_out_ref[...] = sums[...] / cnt
        cprev_out_ref[...] = cur[...]


def _final_body(cprev_ref, cnew_ref, xt_ref, thr_ref, mask_ref, idx_out_ref,
                run_min, run_idx):
    j = pl.program_id(0)

    @pl.when(j == 0)
    def _init():
        run_min[...] = jnp.full_like(run_min, jnp.inf)
        run_idx[...] = jnp.zeros_like(run_idx)

    cp = cprev_ref[...]
    cn = cnew_ref[...]
    xb = xt_ref[...]
    thr = thr_ref[...].reshape(1, _C)
    x2 = jnp.sum(xb * xb, axis=0, keepdims=True)

    c2p = jnp.sum(cp * cp, axis=1, keepdims=True)
    mmp = lax.dot_general(-2.0 * cp, xb, (((1,), (0,)), ((), ())),
                          preferred_element_type=jnp.float32)
    m = (mmp + c2p) < thr
    mask_ref[...] = m

    c2n = jnp.sum(cn * cn, axis=1, keepdims=True)
    mmn = lax.dot_general(-2.0 * cn, xb, (((1,), (0,)), ((), ())),
                          preferred_element_type=jnp.float32)
    d2n = (c2n + x2) + mmn
    disn = jnp.sqrt(jnp.maximum(d2n, 0.0))
    vals = jnp.where(m, disn, jnp.inf)

    bmin = jnp.min(vals, axis=1, keepdims=True)
    li = lax.broadcasted_iota(jnp.int32, vals.shape, 1)
    cand = jnp.where(vals == bmin, li, _C)
    bidx = jnp.min(cand, axis=1, keepdims=True)

    upd = bmin < run_min[...]
    run_idx[...] = jnp.where(upd, j * _C + bidx, run_idx[...])
    run_min[...] = jnp.where(upd, bmin, run_min[...])

    @pl.when(j == _NB - 1)
    def _finish():
        idx_out_ref[...] = run_idx[...]


_GATHER_ROWS_PER_WORKER = 8
_GATHER_WORKERS = _SEED_NUM // _GATHER_ROWS_PER_WORKER


def _sc_gather(x, idx):
    info = plsc.get_sparse_core_info()
    nc = info.num_cores
    mesh = plsc.VectorSubcoreMesh(core_axis_name="c", subcore_axis_name="s")

    @functools.partial(
        pl.kernel, mesh=mesh,
        out_type=jax.ShapeDtypeStruct((_SEED_NUM, _D), jnp.float32),
        compiler_params=pltpu.CompilerParams(use_tc_tiling_on_sc=False),
        scratch_types=[
            pltpu.VMEM((_GATHER_ROWS_PER_WORKER,), jnp.int32),
            pltpu.VMEM((_GATHER_ROWS_PER_WORKER, _D), jnp.float32),
            pltpu.SemaphoreType.DMA,
        ],
    )
    def k(table_hbm, idx_hbm, out_hbm, idx_v, rows_v, sem):
        wid = lax.axis_index("s") * nc + lax.axis_index("c")

        @pl.when(wid < _GATHER_WORKERS)
        def _():
            base = wid * _GATHER_ROWS_PER_WORKER
            pltpu.sync_copy(idx_hbm.at[pl.ds(base, _GATHER_ROWS_PER_WORKER)],
                            idx_v)
            pltpu.async_copy(table_hbm.at[idx_v], rows_v, sem).wait()
            pltpu.sync_copy(rows_v,
                            out_hbm.at[pl.ds(base, _GATHER_ROWS_PER_WORKER)])

    return k(x, idx)


_INIT_IDX = (
    13981, 33398, 10316, 30127, 50841, 5547, 46017, 36849, 44199, 46177,
    20854, 90072, 77379, 30466, 99280, 32312, 27183, 17136, 75016, 1315,
    95086, 46539, 57447, 69504, 37577, 19026, 97387, 60803, 54291, 23894,
    29338, 34337, 4524, 11867, 17076, 63104, 28084, 10117, 89475, 59784,
    25275, 3516, 44150, 87140, 30842, 87331, 77172, 88814, 86999, 78873,
    41737, 78764, 3005, 47461, 20115, 7642, 81396, 74389, 55676, 41898,
    74412, 35131, 46618, 25868,
)


def kernel(x):
    idx0 = jnp.asarray(_INIT_IDX, dtype=jnp.int32)
    centers0 = _sc_gather(x, idx0)

    xt, thr3 = pl.pallas_call(
        _prep_body,
        grid=(_NB,),
        in_specs=[
            pl.BlockSpec((_C, _D), lambda j: (j, 0)),
        ],
        out_specs=[
            pl.BlockSpec((_D, _C), lambda j: (0, j)),
            pl.BlockSpec((1, 1, _C), lambda j: (j, 0, 0)),
        ],
        out_shape=[
            jax.ShapeDtypeStruct((_D, _NPAD), jnp.float32),
            jax.ShapeDtypeStruct((_NB, 1, _C), jnp.float32),
        ],
    )(x)

    centers, centers_prev = pl.pallas_call(
        _iter_body,
        grid=(_ITERATION, _NB),
        in_specs=[
            pl.BlockSpec((_SEED_NUM, _D), lambda it, j: (0, 0)),
            pl.BlockSpec((_D, _C), lambda it, j: (0, j)),
            pl.BlockSpec((1, 1, _C), lambda it, j: (j, 0, 0)),
        ],
        out_specs=[
            pl.BlockSpec((_SEED_NUM, _D), lambda it, j: (0, 0)),
            pl.BlockSpec((_SEED_NUM, _D), lambda it, j: (0, 0)),
        ],
        out_shape=[
            jax.ShapeDtypeStruct((_SEED_NUM, _D), jnp.float32),
            jax.ShapeDtypeStruct((_SEED_NUM, _D), jnp.float32),
        ],
        scratch_shapes=[
            pltpu.VMEM((_SEED_NUM, _D), jnp.float32),
            pltpu.VMEM((_SEED_NUM, _D), jnp.float32),
            pltpu.VMEM((_SEED_NUM, 1), jnp.float32),
        ],
    )(centers0, xt, thr3)

    mask, idx2d = pl.pallas_call(
        _final_body,
        grid=(_NB,),
        in_specs=[
            pl.BlockSpec((_SEED_NUM, _D), lambda j: (0, 0)),
            pl.BlockSpec((_SEED_NUM, _D), lambda j: (0, 0)),
            pl.BlockSpec((_D, _C), lambda j: (0, j)),
            pl.BlockSpec((1, 1, _C), lambda j: (j, 0, 0)),
        ],
        out_specs=[
            pl.BlockSpec((_SEED_NUM, _C), lambda j: (0, j)),
            pl.BlockSpec((_SEED_NUM, 1), lambda j: (0, 0)),
        ],
        out_shape=[
            jax.ShapeDtypeStruct((_SEED_NUM, _N), jnp.bool_),
            jax.ShapeDtypeStruct((_SEED_NUM, 1), jnp.int32),
        ],
        scratch_shapes=[
            pltpu.VMEM((_SEED_NUM, 1), jnp.float32),
            pltpu.VMEM((_SEED_NUM, 1), jnp.int32),
        ],
    )(centers_prev, centers, xt, thr3)

    index_similar = idx2d.reshape(_SEED_NUM)
    center_similar = _sc_gather(x, index_similar)
    return centers, mask, center_similar, index_similar


# speedup vs baseline: 2.4098x; 1.0591x over previous
import functools

import jax
import jax.numpy as jnp
from jax import lax
from jax.experimental import pallas as pl
from jax.experimental.pallas import tpu as pltpu
from jax.experimental.pallas import tpu_sc as plsc

_SEED_NUM = 64
_BANDWIDTH = 11.5
_BW2 = _BANDWIDTH * _BANDWIDTH
_ITERATION = 5

_N = 100000
_D = 64
_C = 12800
_NB = 8
_NPAD = _NB * _C
_PAD_VAL = 1.0e4


def _fused_body(x_ref, c0_ref, c_out_ref, mask_ref, idx_out_ref,
                xt_s, thr_s, cur, prev, sums, counts, run_min, run_idx):
    it = pl.program_id(0)
    j = pl.program_id(1)

    @pl.when(jnp.logical_and(it == 0, j == 0))
    def _init():
        cur[...] = c0_ref[...]
        sums[...] = jnp.zeros_like(sums)
        counts[...] = jnp.zeros_like(counts)

    @pl.when(jnp.logical_and(it > 0, j == 0))
    def _advance():
        cnt = jnp.maximum(counts[...], 1.0)
        prev[...] = cur[...]
        cur[...] = sums[...] / cnt
        sums[...] = jnp.zeros_like(sums)
        counts[...] = jnp.zeros_like(counts)

    @pl.when(jnp.logical_and(it == _ITERATION, j == 0))
    def _init_argmin():
        run_min[...] = jnp.full_like(run_min, jnp.inf)
        run_idx[...] = jnp.zeros_like(run_idx)

    @pl.when(it == 0)
    def _load_transpose():
        raw = x_ref[...]
        xt = raw.T
        lane = lax.broadcasted_iota(jnp.int32, (1, _C), 1)
        valid = lane < (_N - j * _C)
        xt = jnp.where(valid, xt, _PAD_VAL)
        xt_s[:, pl.ds(j * _C, _C)] = xt
        x2 = jnp.sum(xt * xt, axis=0, keepdims=True)
        thr_s[:, pl.ds(j * _C, _C)] = _BW2 - x2

    xb = xt_s[:, pl.ds(j * _C, _C)]
    thr = thr_s[:, pl.ds(j * _C, _C)]

    @pl.when(it < _ITERATION)
    def _accumulate():
        c = cur[...]
        c2 = jnp.sum(c * c, axis=1, keepdims=True)
        mm = lax.dot_general(-2.0 * c, xb, (((1,), (0,)), ((), ())),
                             preferred_element_type=jnp.float32)
        maskf = ((mm + c2) < thr).astype(jnp.float32)
        sums[...] += lax.dot_general(maskf, xb, (((1,), (1,)), ((), ())),
                                     preferred_element_type=jnp.float32)
        ones_col = jnp.ones((_C, 1), dtype=jnp.float32)
        counts[...] += lax.dot_general(maskf, ones_col,
                                       (((1,), (0,)), ((), ())),
                                       preferred_element_type=jnp.float32)

    @pl.when(it == _ITERATION)
    def _finalize():
        cp = prev[...]
        cn = cur[...]
        x2 = jnp.sum(xb * xb, axis=0, keepdims=True)

        c2p = jnp.sum(cp * cp, axis=1, keepdims=True)
        mmp = lax.dot_general(-2.0 * cp, xb, (((1,), (0,)), ((), ())),
                              preferred_element_type=jnp.float32)
        m = (mmp + c2p) < thr
        mask_ref[...] = m

        c2n = jnp.sum(cn * cn, axis=1, keepdims=True)
        mmn = lax.dot_general(-2.0 * cn, xb, (((1,), (0,)), ((), ())),
                              preferred_element_type=jnp.float32)
        d2n = (c2n + x2) + mmn
        disn = jnp.sqrt(jnp.maximum(d2n, 0.0))
        vals = jnp.where(m, disn, jnp.inf)

        bmin = jnp.min(vals, axis=1, keepdims=True)
        li = lax.broadcasted_iota(jnp.int32, vals.shape, 1)
        cand = jnp.where(vals == bmin, li, _C)
        bidx = jnp.min(cand, axis=1, keepdims=True)

        upd = bmin < run_min[...]
        run_idx[...] = jnp.where(upd, j * _C + bidx, run_idx[...])
        run_min[...] = jnp.where(upd, bmin, run_min[...])

        @pl.when(j == _NB - 1)
        def _emit():
            c_out_ref[...] = cn
            idx_out_ref[...] = run_idx[...]


_GATHER_ROWS_PER_WORKER = 8
_GATHER_WORKERS = _SEED_NUM // _GATHER_ROWS_PER_WORKER


def _sc_gather(x, idx):
    info = plsc.get_sparse_core_info()
    nc = info.num_cores
    mesh = plsc.VectorSubcoreMesh(core_axis_name="c", subcore_axis_name="s")

    @functools.partial(
        pl.kernel, mesh=mesh,
        out_type=jax.ShapeDtypeStruct((_SEED_NUM, _D), jnp.float32),
        compiler_params=pltpu.CompilerParams(use_tc_tiling_on_sc=False),
        scratch_types=[
            pltpu.VMEM((_GATHER_ROWS_PER_WORKER,), jnp.int32),
            pltpu.VMEM((_GATHER_ROWS_PER_WORKER, _D), jnp.float32),
            pltpu.SemaphoreType.DMA,
        ],
    )
    def k(table_hbm, idx_hbm, out_hbm, idx_v, rows_v, sem):
        wid = lax.axis_index("s") * nc + lax.axis_index("c")

        @pl.when(wid < _GATHER_WORKERS)
        def _():
            base = wid * _GATHER_ROWS_PER_WORKER
            pltpu.sync_copy(idx_hbm.at[pl.ds(base, _GATHER_ROWS_PER_WORKER)],
                            idx_v)
            pltpu.async_copy(table_hbm.at[idx_v], rows_v, sem).wait()
            pltpu.sync_copy(rows_v,
                            out_hbm.at[pl.ds(base, _GATHER_ROWS_PER_WORKER)])

    return k(x, idx)


_INIT_IDX = (
    13981, 33398, 10316, 30127, 50841, 5547, 46017, 36849, 44199, 46177,
    20854, 90072, 77379, 30466, 99280, 32312, 27183, 17136, 75016, 1315,
    95086, 46539, 57447, 69504, 37577, 19026, 97387, 60803, 54291, 23894,
    29338, 34337, 4524, 11867, 17076, 63104, 28084, 10117, 89475, 59784,
    25275, 3516, 44150, 87140, 30842, 87331, 77172, 88814, 86999, 78873,
    41737, 78764, 3005, 47461, 20115, 7642, 81396, 74389, 55676, 41898,
    74412, 35131, 46618, 25868,
)


def kernel(x):
    idx0 = jnp.asarray(_INIT_IDX, dtype=jnp.int32)
    centers0 = _sc_gather(x, idx0)

    centers, mask, idx2d = pl.pallas_call(
        _fused_body,
        grid=(_ITERATION + 1, _NB),
        in_specs=[
            pl.BlockSpec((_C, _D),
                         lambda it, j: (jnp.where(it == 0, j, 0), 0)),
            pl.BlockSpec((_SEED_NUM, _D), lambda it, j: (0, 0)),
        ],
        out_specs=[
            pl.BlockSpec((_SEED_NUM, _D), lambda it, j: (0, 0)),
            pl.BlockSpec((_SEED_NUM, _C),
                         lambda it, j: (0, jnp.where(it == _ITERATION, j, 0))),
            pl.BlockSpec((_SEED_NUM, 1), lambda it, j: (0, 0)),
        ],
        out_shape=[
            jax.ShapeDtypeStruct((_SEED_NUM, _D), jnp.float32),
            jax.ShapeDtypeStruct((_SEED_NUM, _N), jnp.bool_),
            jax.ShapeDtypeStruct((_SEED_NUM, 1), jnp.int32),
        ],
        scratch_shapes=[
            pltpu.VMEM((_D, _NPAD), jnp.float32),
            pltpu.VMEM((1, _NPAD), jnp.float32),
            pltpu.VMEM((_SEED_NUM, _D), jnp.float32),
            pltpu.VMEM((_SEED_NUM, _D), jnp.float32),
            pltpu.VMEM((_SEED_NUM, _D), jnp.float32),
            pltpu.VMEM((_SEED_NUM, 1), jnp.float32),
            pltpu.VMEM((_SEED_NUM, 1), jnp.float32),
            pltpu.VMEM((_SEED_NUM, 1), jnp.int32),
        ],
        compiler_params=pltpu.CompilerParams(
            vmem_limit_bytes=100 * 1024 * 1024,
        ),
    )(x, centers0)

    index_similar = idx2d.reshape(_SEED_NUM)
    center_similar = _sc_gather(x, index_similar)
    return centers, mask, center_similar, index_similar
